# baseline (speedup 1.0000x reference)
import jax
import jax.numpy as jnp
from jax import lax
from jax.experimental import pallas as pl
from jax.experimental.pallas import tpu as pltpu
from jax.experimental.pallas import tpu_sc as plsc

B, S = 1024, 200
BS = B * S
INTD = 64
GW = 128
HD = 192
REP = 64


_NC, _NS = 2, 16
_NW = _NC * _NS
_PER_W = BS // _NW
_C = 64
_NCHUNK = _PER_W // _C


def _sc_gather_body(idx1, idx2, idx3, t1, t2, t3,
                    o1, o2, o3, iv1, iv2, iv3,
                    ea1, ea2, ea3, eb1, eb2, eb3,
                    pa1, pa2, pa3, pb1, pb2, pb3,
                    sema, semb):
    wid = lax.axis_index("s") * _NC + lax.axis_index("c")
    base0 = wid * _PER_W
    tabs = (t1, t2, t3)
    ivs = (iv1, iv2, iv3)
    outs = (o1, o2, o3)
    bufs = ((ea1, ea2, ea3), (eb1, eb2, eb3))
    pbufs = ((pa1, pa2, pa3), (pb1, pb2, pb3))
    sems = (sema, semb)

    pltpu.sync_copy(idx1.at[pl.ds(base0, _PER_W)], iv1)
    pltpu.sync_copy(idx2.at[pl.ds(base0, _PER_W)], iv2)
    pltpu.sync_copy(idx3.at[pl.ds(base0, _PER_W)], iv3)

    def fire(g, s):
        for j in range(3):
            pltpu.async_copy(tabs[j].at[ivs[j].at[pl.ds(g * _C, _C)]],
                             bufs[s][j], sems[s])

    def drain(s):
        for j in range(3):
            pltpu.make_async_copy(tabs[j].at[pl.ds(0, _C)],
                                  bufs[s][j], sems[s]).wait()

    def repack(s):
        def row(r, _):
            for j in range(3):
                for q in range(INTD // 16):
                    pbufs[s][j][r, pl.ds(16 * q, 16)] = (
                        bufs[s][j][r, pl.ds(16 * q, 16)])
            return ()
        lax.fori_loop(0, _C, row, (), unroll=2)

    def scatter(g, s):
        base = base0 + g * _C
        for j in range(3):
            pltpu.sync_copy(pbufs[s][j], outs[j].at[pl.ds(base, _C)])

    fire(0, 0)

    def pair(k, _):
        g = 2 * k
        fire(g + 1, 1)
        drain(0)
        repack(0)
        scatter(g, 0)
        fire(g + 2, 0)
        drain(1)
        repack(1)
        scatter(g + 1, 1)
        return ()

    lax.fori_loop(0, _NCHUNK // 2 - 1, pair, (), unroll=False)
    g = _NCHUNK - 2
    fire(g + 1, 1)
    drain(0)
    repack(0)
    scatter(g, 0)
    drain(1)
    repack(1)
    scatter(g + 1, 1)


def _sc_gather(idx1, idx2, idx3, t1, t2, t3):
    mesh = plsc.VectorSubcoreMesh(core_axis_name="c", subcore_axis_name="s")
    row = jax.ShapeDtypeStruct((BS, INTD), jnp.float32)
    ebuf = pltpu.VMEM((_C, GW), jnp.float32)
    pbuf = pltpu.VMEM((_C, INTD), jnp.float32)
    f = pl.kernel(
        _sc_gather_body,
        mesh=mesh,
        out_type=(row, row, row),
        scratch_types=[
            pltpu.VMEM((_PER_W,), jnp.int32),
            pltpu.VMEM((_PER_W,), jnp.int32),
            pltpu.VMEM((_PER_W,), jnp.int32),
            ebuf, ebuf, ebuf, ebuf, ebuf, ebuf,
            pbuf, pbuf, pbuf, pbuf, pbuf, pbuf,
            pltpu.SemaphoreType.DMA,
            pltpu.SemaphoreType.DMA,
        ],
    )
    return f(idx1, idx2, idx3, t1, t2, t3)


_R = 2048


def _tc_body(c1_ref, c2_ref, c3_ref, i0_ref, el_ref, du_ref, emb0_ref,
             w0_ref, w_ref, wel_ref, wdu_ref, b_ref, out_ref):
    w = w_ref[...]
    acc = jnp.dot(c1_ref[...], w[0 * INTD:1 * INTD],
                  preferred_element_type=jnp.float32)
    acc += jnp.dot(c2_ref[...], w[1 * INTD:2 * INTD],
                   preferred_element_type=jnp.float32)
    acc += jnp.dot(c3_ref[...], w[2 * INTD:3 * INTD],
                   preferred_element_type=jnp.float32)
    m0 = jnp.dot(emb0_ref[...], w0_ref[...],
                 preferred_element_type=jnp.float32)
    iota8 = lax.broadcasted_iota(jnp.int32, (1, 8), 1)
    oh = jnp.where(i0_ref[...][:, None] == iota8, 1.0, 0.0)
    acc += jnp.dot(oh, m0, preferred_element_type=jnp.float32)
    el = el_ref[...][:, None]
    du = du_ref[...][:, None]
    out_ref[...] = (acc + el * wel_ref[...][None, :] + du * wdu_ref[...][None, :]
                    + b_ref[...][None, :])


def _tc_matmul(c1, c2, c3, i0, el, du, emb0, w0, w_mid, w_el, w_du, b):
    grid = (BS // _R,)
    row_spec = pl.BlockSpec((_R, INTD), lambda i: (i, 0))
    flat_spec = pl.BlockSpec((_R,), lambda i: (i,))
    return pl.pallas_call(
        _tc_body,
        grid=grid,
        in_specs=[
            row_spec, row_spec, row_spec,
            flat_spec, flat_spec, flat_spec,
            pl.BlockSpec((8, INTD), lambda i: (0, 0)),
            pl.BlockSpec((INTD, HD), lambda i: (0, 0)),
            pl.BlockSpec((3 * INTD, HD), lambda i: (0, 0)),
            pl.BlockSpec((HD,), lambda i: (0,)),
            pl.BlockSpec((HD,), lambda i: (0,)),
            pl.BlockSpec((HD,), lambda i: (0,)),
        ],
        out_specs=pl.BlockSpec((_R, HD), lambda i: (i, 0)),
        out_shape=jax.ShapeDtypeStruct((BS, HD), jnp.float32),
    )(c1, c2, c3, i0, el, du, emb0, w0, w_mid, w_el, w_du, b)


def kernel(interaction, assessmentItemID, testId, KnowledgeTag, elapsed,
           duration, emb_interaction, emb_assessmentItemID, emb_testId,
           emb_KnowledgeTag, W, b):
    batch_size, seq_len = interaction.shape[0], interaction.shape[1]
    zcol = jnp.zeros((100001, GW - INTD), jnp.float32)
    t1 = jnp.concatenate([emb_assessmentItemID, zcol], axis=1)
    rep = lambda t: jnp.tile(jnp.pad(t, ((0, 1024 - 1001), (0, GW - INTD))),
                             (REP, 1))
    t2 = rep(emb_testId)
    t3 = rep(emb_KnowledgeTag)
    iota = jnp.arange(BS, dtype=jnp.int32)
    spread = (iota & (REP - 1)) << 10
    c1, c2, c3 = _sc_gather(
        assessmentItemID.reshape(-1), testId.reshape(-1) + spread,
        KnowledgeTag.reshape(-1) + spread, t1, t2, t3)
    emb0 = jnp.pad(emb_interaction, ((0, 8 - 3), (0, 0)))
    X = _tc_matmul(c1, c2, c3, interaction.reshape(-1), elapsed.reshape(-1),
                   duration.reshape(-1), emb0, W[:INTD], W[INTD:4 * INTD],
                   W[4 * INTD], W[4 * INTD + 1], b)
    return (X.reshape(batch_size, seq_len, HD), batch_size, seq_len)


# speedup vs baseline: 1.2840x; 1.2840x over previous
import jax
import jax.numpy as jnp
from jax import lax
from jax.experimental import pallas as pl
from jax.experimental.pallas import tpu as pltpu
from jax.experimental.pallas import tpu_sc as plsc

B, S = 1024, 200
BS = B * S
HALF = BS // 2
INTD = 64
GW = 128
HD = 192
REP = 64


_NC, _NS = 2, 16
_NW = _NC * _NS
_PER_W = HALF // _NW
_C = 64
_NCHUNK = _PER_W // _C


def _make_sc_body(half):
    def body(idx1, idx2, idx3, t1, t2, t3,
             o1, o2, o3, iv1, iv2, iv3,
             ea1, ea2, ea3, eb1, eb2, eb3,
             sema, semb):
        wid = lax.axis_index("s") * _NC + lax.axis_index("c")
        src0 = half * HALF + wid * _PER_W
        dst0 = wid * _PER_W
        tabs = (t1, t2, t3)
        ivs = (iv1, iv2, iv3)
        outs = (o1, o2, o3)
        bufs = ((ea1, ea2, ea3), (eb1, eb2, eb3))
        sems = (sema, semb)

        pltpu.sync_copy(idx1.at[pl.ds(src0, _PER_W)], iv1)
        pltpu.sync_copy(idx2.at[pl.ds(src0, _PER_W)], iv2)
        pltpu.sync_copy(idx3.at[pl.ds(src0, _PER_W)], iv3)

        def fire(g, s):
            for j in range(3):
                pltpu.async_copy(tabs[j].at[ivs[j].at[pl.ds(g * _C, _C)]],
                                 bufs[s][j], sems[s])

        def drain(s):
            for j in range(3):
                pltpu.make_async_copy(tabs[j].at[pl.ds(0, _C)],
                                      bufs[s][j], sems[s]).wait()

        def scatter(g, s):
            base = dst0 + g * _C
            for j in range(3):
                pltpu.sync_copy(bufs[s][j], outs[j].at[pl.ds(base, _C)])

        fire(0, 0)

        def pair(k, _):
            g = 2 * k
            fire(g + 1, 1)
            drain(0)
            scatter(g, 0)
            fire(g + 2, 0)
            drain(1)
            scatter(g + 1, 1)
            return ()

        lax.fori_loop(0, _NCHUNK // 2 - 1, pair, (), unroll=False)
        g = _NCHUNK - 2
        fire(g + 1, 1)
        drain(0)
        scatter(g, 0)
        drain(1)
        scatter(g + 1, 1)

    return body


def _sc_gather(half, idx1, idx2, idx3, t1, t2, t3):
    mesh = plsc.VectorSubcoreMesh(core_axis_name="c", subcore_axis_name="s")
    row = jax.ShapeDtypeStruct((HALF, GW), jnp.float32)
    ebuf = pltpu.VMEM((_C, GW), jnp.float32)
    f = pl.kernel(
        _make_sc_body(half),
        mesh=mesh,
        out_type=(row, row, row),
        scratch_types=[
            pltpu.VMEM((_PER_W,), jnp.int32),
            pltpu.VMEM((_PER_W,), jnp.int32),
            pltpu.VMEM((_PER_W,), jnp.int32),
            ebuf, ebuf, ebuf, ebuf, ebuf, ebuf,
            pltpu.SemaphoreType.DMA,
            pltpu.SemaphoreType.DMA,
        ],
    )
    return f(idx1, idx2, idx3, t1, t2, t3)


_R = 2048
_NBLK = HALF // _R


def _tc_body(c1_ref, c2_ref, c3_ref, i0_ref, el_ref, du_ref, emb0_ref,
             w0_ref, w_ref, wel_ref, wdu_ref, b_ref, *rest):
    out_ref = rest[-1]
    w = w_ref[...]
    acc = jnp.dot(c1_ref[...], w[0 * GW:1 * GW],
                  preferred_element_type=jnp.float32)
    acc += jnp.dot(c2_ref[...], w[1 * GW:2 * GW],
                   preferred_element_type=jnp.float32)
    acc += jnp.dot(c3_ref[...], w[2 * GW:3 * GW],
                   preferred_element_type=jnp.float32)
    m0 = jnp.dot(emb0_ref[...], w0_ref[...],
                 preferred_element_type=jnp.float32)
    iota8 = lax.broadcasted_iota(jnp.int32, (1, 8), 1)
    oh = jnp.where(i0_ref[...][:, None] == iota8, 1.0, 0.0)
    acc += jnp.dot(oh, m0, preferred_element_type=jnp.float32)
    el = el_ref[...][:, None]
    du = du_ref[...][:, None]
    out_ref[...] = (acc + el * wel_ref[...][None, :] + du * wdu_ref[...][None, :]
                    + b_ref[...][None, :])


def _tc_matmul(half, c1, c2, c3, i0, el, du, emb0, w0, w_mid, w_el, w_du, b,
               xprev=None):
    off = half * _NBLK
    row_spec = pl.BlockSpec((_R, GW), lambda i: (i, 0))
    flat_spec = pl.BlockSpec((_R,), lambda i, off=off: (i + off,))
    in_specs = [
        row_spec, row_spec, row_spec,
        flat_spec, flat_spec, flat_spec,
        pl.BlockSpec((8, INTD), lambda i: (0, 0)),
        pl.BlockSpec((INTD, HD), lambda i: (0, 0)),
        pl.BlockSpec((3 * GW, HD), lambda i: (0, 0)),
        pl.BlockSpec((HD,), lambda i: (0,)),
        pl.BlockSpec((HD,), lambda i: (0,)),
        pl.BlockSpec((HD,), lambda i: (0,)),
    ]
    args = [c1, c2, c3, i0, el, du, emb0, w0, w_mid, w_el, w_du, b]
    aliases = {}
    if xprev is not None:
        in_specs.append(pl.BlockSpec(memory_space=pl.ANY))
        args.append(xprev)
        aliases = {12: 0}
    return pl.pallas_call(
        _tc_body,
        grid=(_NBLK,),
        in_specs=in_specs,
        out_specs=pl.BlockSpec((_R, HD), lambda i, off=off: (i + off, 0)),
        out_shape=jax.ShapeDtypeStruct((BS, HD), jnp.float32),
        input_output_aliases=aliases,
    )(*args)


def kernel(interaction, assessmentItemID, testId, KnowledgeTag, elapsed,
           duration, emb_interaction, emb_assessmentItemID, emb_testId,
           emb_KnowledgeTag, W, b):
    batch_size, seq_len = interaction.shape[0], interaction.shape[1]
    zcol = jnp.zeros((100001, GW - INTD), jnp.float32)
    t1 = jnp.concatenate([emb_assessmentItemID, zcol], axis=1)
    rep = lambda t: jnp.tile(jnp.pad(t, ((0, 1024 - 1001), (0, GW - INTD))),
                             (REP, 1))
    t2 = rep(emb_testId)
    t3 = rep(emb_KnowledgeTag)
    iota = jnp.arange(BS, dtype=jnp.int32)
    spread = (iota & (REP - 1)) << 10
    i1 = assessmentItemID.reshape(-1)
    i2 = testId.reshape(-1) + spread
    i3 = KnowledgeTag.reshape(-1) + spread
    ca = _sc_gather(0, i1, i2, i3, t1, t2, t3)
    cb = _sc_gather(1, i1, i2, i3, t1, t2, t3)
    emb0 = jnp.pad(emb_interaction, ((0, 8 - 3), (0, 0)))
    w_pad = jnp.concatenate(
        [W[INTD:4 * INTD].reshape(3, INTD, HD),
         jnp.zeros((3, GW - INTD, HD), jnp.float32)], axis=1).reshape(3 * GW, HD)
    i0 = interaction.reshape(-1)
    el = elapsed.reshape(-1)
    du = duration.reshape(-1)
    common = (i0, el, du, emb0, W[:INTD], w_pad,
              W[4 * INTD], W[4 * INTD + 1], b)
    xa = _tc_matmul(0, *ca, *common)
    x = _tc_matmul(1, *cb, *common, xprev=xa)
    return (x.reshape(batch_size, seq_len, HD), batch_size, seq_len)
